# baseline (speedup 1.0000x reference)
import functools

import jax
import jax.numpy as jnp
from jax import lax
from jax.experimental import pallas as pl
from jax.experimental.pallas import tpu as pltpu
from jax.experimental.pallas import tpu_sc as plsc

N = 10000
E = 320000
D = 128
OUT = 10

NC = 2
NS = 16
NW = NC * NS
CSZ = 112
CH0 = 96
CH1 = 96
TOTCH = NS * (CH0 + CH1)
EPAD = TOTCH * CSZ
NPAD = 10112
RPW = NPAD // NS

G = 12
GPW = CH0 // G
NG = TOTCH // G


def _msg_body(h_hbm, idx_hbm, z_hbm, out_hbm,
              idx_a, idx_b, rows_a, rows_b, rows_c, table,
              sem_ia, sem_ib, sem_a, sem_b, sem_c):
  cid = lax.axis_index("c")
  sid = lax.axis_index("s")
  r0 = sid * RPW
  base = (cid * NS + sid) * GPW
  pltpu.sync_copy(z_hbm.at[pl.ds(r0, RPW)], table.at[pl.ds(r0, RPW)])
  plsc.subcore_barrier()

  pltpu.sync_copy(idx_hbm.at[base], idx_a)
  pltpu.async_copy(idx_hbm.at[base + 1], idx_b, sem_ib)

  def process(buf):
    pltpu.async_copy(h_hbm.at[buf.at[0]], rows_a, sem_a)
    pltpu.async_copy(h_hbm.at[buf.at[1]], rows_b, sem_b)

    @pl.loop(0, G, step=3)
    def _(i):
      pltpu.make_async_copy(h_hbm.at[buf.at[i]], rows_a, sem_a).wait()

      @pl.when(i + 2 < G)
      def _():
        pltpu.async_copy(h_hbm.at[buf.at[i + 2]], rows_c, sem_c)

      pltpu.sync_copy(rows_a, table.at[buf.at[G + i]], add=True)

      pltpu.make_async_copy(h_hbm.at[buf.at[i + 1]], rows_b, sem_b).wait()

      @pl.when(i + 3 < G)
      def _():
        pltpu.async_copy(h_hbm.at[buf.at[i + 3]], rows_a, sem_a)

      pltpu.sync_copy(rows_b, table.at[buf.at[G + i + 1]], add=True)

      pltpu.make_async_copy(h_hbm.at[buf.at[i + 2]], rows_c, sem_c).wait()

      @pl.when(i + 4 < G)
      def _():
        pltpu.async_copy(h_hbm.at[buf.at[i + 4]], rows_b, sem_b)

      pltpu.sync_copy(rows_c, table.at[buf.at[G + i + 2]], add=True)

  @pl.loop(0, GPW, step=2)
  def _(g):
    process(idx_a)

    @pl.when(g + 2 < GPW)
    def _():
      pltpu.async_copy(idx_hbm.at[base + g + 2], idx_a, sem_ia)

    pltpu.make_async_copy(idx_hbm.at[base + g + 1], idx_b, sem_ib).wait()
    process(idx_b)

    @pl.when(g + 3 < GPW)
    def _():
      pltpu.async_copy(idx_hbm.at[base + g + 3], idx_b, sem_ib)

    @pl.when(g + 2 < GPW)
    def _():
      pltpu.make_async_copy(idx_hbm.at[base + g + 2], idx_a, sem_ia).wait()

  plsc.subcore_barrier()
  pltpu.sync_copy(table.at[pl.ds(r0, RPW)], out_hbm.at[cid, pl.ds(r0, RPW)])


@functools.lru_cache(maxsize=1)
def _get_sc_message_pass():
  mesh = plsc.VectorSubcoreMesh(core_axis_name="c", subcore_axis_name="s",
                                num_cores=NC, num_subcores=NS)
  return functools.partial(
      pl.kernel,
      out_type=jax.ShapeDtypeStruct((NC, NPAD, D), jnp.float32),
      mesh=mesh,
      scratch_types=[
          pltpu.VMEM((2 * G, CSZ), jnp.int32),
          pltpu.VMEM((2 * G, CSZ), jnp.int32),
          pltpu.VMEM((CSZ, D), jnp.float32),
          pltpu.VMEM((CSZ, D), jnp.float32),
          pltpu.VMEM((CSZ, D), jnp.float32),
          pltpu.VMEM_SHARED((NPAD, D), jnp.float32),
          pltpu.SemaphoreType.DMA,
          pltpu.SemaphoreType.DMA,
          pltpu.SemaphoreType.DMA,
          pltpu.SemaphoreType.DMA,
          pltpu.SemaphoreType.DMA,
      ],
  )(_msg_body)


R = 1000


def _mlp_block(h_ref, p0_ref, p1_ref, w1_ref, b1_ref, w2_ref, b2_ref):
  agg = h_ref[...] + p0_ref[0] + p1_ref[0]
  z = jnp.dot(agg, w1_ref[...], preferred_element_type=jnp.float32)
  z = jnp.maximum(z + b1_ref[...], 0.0)
  o = jnp.dot(z, w2_ref[...], preferred_element_type=jnp.float32)
  return jnp.maximum(o + b2_ref[...], 0.0)


def _mlp_body(h_ref, p0_ref, p1_ref, w1_ref, b1_ref, w2_ref, b2_ref, o_ref):
  o_ref[...] = _mlp_block(h_ref, p0_ref, p1_ref, w1_ref, b1_ref, w2_ref,
                          b2_ref)


def _mlp_last_body(h_ref, p0_ref, p1_ref, w1_ref, b1_ref, w2_ref, b2_ref,
                   wo_ref, bo_ref, o_ref, gmax_ref):
  i = pl.program_id(0)
  o = _mlp_block(h_ref, p0_ref, p1_ref, w1_ref, b1_ref, w2_ref, b2_ref)
  bm = jnp.max(o, axis=0, keepdims=True)

  @pl.when(i == 0)
  def _():
    gmax_ref[...] = bm

  @pl.when(i > 0)
  def _():
    gmax_ref[...] = jnp.maximum(gmax_ref[...], bm)

  @pl.when(i == pl.num_programs(0) - 1)
  def _():
    g = gmax_ref[...]
    o_ref[...] = (jnp.dot(g, wo_ref[...], preferred_element_type=jnp.float32)
                  + bo_ref[...])


_p_spec0 = pl.BlockSpec((1, R, D), lambda i: (0, i, 0))
_p_spec1 = pl.BlockSpec((1, R, D), lambda i: (1, i, 0))
_h_spec = pl.BlockSpec((R, D), lambda i: (i, 0))
_w1_spec = pl.BlockSpec((D, 2 * D), lambda i: (0, 0))
_b1_spec = pl.BlockSpec((1, 2 * D), lambda i: (0, 0))
_w2_spec = pl.BlockSpec((2 * D, D), lambda i: (0, 0))
_b2_spec = pl.BlockSpec((1, D), lambda i: (0, 0))

_tc_mlp = pl.pallas_call(
    _mlp_body,
    grid=(N // R,),
    in_specs=[_h_spec, _p_spec0, _p_spec1, _w1_spec, _b1_spec, _w2_spec,
              _b2_spec],
    out_specs=_h_spec,
    out_shape=jax.ShapeDtypeStruct((N, D), jnp.float32),
)

_tc_mlp_last = pl.pallas_call(
    _mlp_last_body,
    grid=(N // R,),
    in_specs=[_h_spec, _p_spec0, _p_spec1, _w1_spec, _b1_spec, _w2_spec,
              _b2_spec,
              pl.BlockSpec((D, OUT), lambda i: (0, 0)),
              pl.BlockSpec((1, OUT), lambda i: (0, 0))],
    out_specs=pl.BlockSpec((1, OUT), lambda i: (0, 0)),
    out_shape=jax.ShapeDtypeStruct((1, OUT), jnp.float32),
    scratch_shapes=[pltpu.VMEM((1, D), jnp.float32)],
)


def kernel(x, edge_index, W1_0, b1_0, W2_0, b2_0, W1_1, b1_1, W2_1, b2_1,
           W1_2, b1_2, W2_2, b2_2, W_out, b_out):
  src = edge_index[0]
  dst = edge_index[1]
  pad = EPAD - E
  pad_iota = jnp.arange(pad, dtype=jnp.int32)
  srcp = jnp.concatenate([src, pad_iota % N])
  dstp = jnp.concatenate([dst, N + pad_iota % (NPAD - N)])
  idxp = jnp.concatenate(
      [srcp.reshape(NG, G, CSZ), dstp.reshape(NG, G, CSZ)], axis=1)
  zeros = jnp.zeros((NPAD, D), jnp.float32)

  params = [(W1_0, b1_0, W2_0, b2_0), (W1_1, b1_1, W2_1, b2_1),
            (W1_2, b1_2, W2_2, b2_2)]
  sc_message_pass = _get_sc_message_pass()
  h = x
  for l, (W1, b1, W2, b2) in enumerate(params):
    p = sc_message_pass(h, idxp, zeros)
    if l < 2:
      h = _tc_mlp(h, p, p, W1, b1.reshape(1, -1), W2, b2.reshape(1, -1))
    else:
      out = _tc_mlp_last(h, p, p, W1, b1.reshape(1, -1), W2,
                         b2.reshape(1, -1), W_out, b_out.reshape(1, -1))
  return out


# speedup vs baseline: 1.0225x; 1.0225x over previous
import functools

import jax
import jax.numpy as jnp
from jax import lax
from jax.experimental import pallas as pl
from jax.experimental.pallas import tpu as pltpu
from jax.experimental.pallas import tpu_sc as plsc

N = 10000
E = 320000
D = 128
OUT = 10

NC = 2
NS = 16
NW = NC * NS
CSZ = 112
CH0 = 96
CH1 = 96
TOTCH = NS * (CH0 + CH1)
EPAD = TOTCH * CSZ
NPAD = 10112
RPW = NPAD // NS

G = 12
GPW = CH0 // G
NG = TOTCH // G


def _msg_body(h_hbm, idx_hbm, z_hbm, out_hbm,
              idx_a, idx_b, rows_a, rows_b, rows_c, table,
              sem_ia, sem_ib, sem_a, sem_b, sem_c):
  cid = lax.axis_index("c")
  sid = lax.axis_index("s")
  r0 = sid * RPW
  base = (cid * NS + sid) * GPW
  @pl.when((cid == 0) & (sid < NS - 1))
  def _():
    pltpu.sync_copy(h_hbm.at[pl.ds(r0, RPW)], table.at[pl.ds(r0, RPW)])

  @pl.when((cid == 0) & (sid == NS - 1))
  def _():
    pltpu.sync_copy(h_hbm.at[pl.ds(r0, N - r0)], table.at[pl.ds(r0, N - r0)])
    pltpu.sync_copy(z_hbm.at[pl.ds(N, NPAD - N)], table.at[pl.ds(N, NPAD - N)])

  @pl.when(cid == 1)
  def _():
    pltpu.sync_copy(z_hbm.at[pl.ds(r0, RPW)], table.at[pl.ds(r0, RPW)])

  plsc.subcore_barrier()

  pltpu.sync_copy(idx_hbm.at[base], idx_a)
  pltpu.async_copy(idx_hbm.at[base + 1], idx_b, sem_ib)

  def process(buf):
    pltpu.async_copy(h_hbm.at[buf.at[0]], rows_a, sem_a)
    pltpu.async_copy(h_hbm.at[buf.at[1]], rows_b, sem_b)

    @pl.loop(0, G, step=3)
    def _(i):
      pltpu.make_async_copy(h_hbm.at[buf.at[i]], rows_a, sem_a).wait()

      @pl.when(i + 2 < G)
      def _():
        pltpu.async_copy(h_hbm.at[buf.at[i + 2]], rows_c, sem_c)

      pltpu.sync_copy(rows_a, table.at[buf.at[G + i]], add=True)

      pltpu.make_async_copy(h_hbm.at[buf.at[i + 1]], rows_b, sem_b).wait()

      @pl.when(i + 3 < G)
      def _():
        pltpu.async_copy(h_hbm.at[buf.at[i + 3]], rows_a, sem_a)

      pltpu.sync_copy(rows_b, table.at[buf.at[G + i + 1]], add=True)

      pltpu.make_async_copy(h_hbm.at[buf.at[i + 2]], rows_c, sem_c).wait()

      @pl.when(i + 4 < G)
      def _():
        pltpu.async_copy(h_hbm.at[buf.at[i + 4]], rows_b, sem_b)

      pltpu.sync_copy(rows_c, table.at[buf.at[G + i + 2]], add=True)

  @pl.loop(0, GPW, step=2)
  def _(g):
    process(idx_a)

    @pl.when(g + 2 < GPW)
    def _():
      pltpu.async_copy(idx_hbm.at[base + g + 2], idx_a, sem_ia)

    pltpu.make_async_copy(idx_hbm.at[base + g + 1], idx_b, sem_ib).wait()
    process(idx_b)

    @pl.when(g + 3 < GPW)
    def _():
      pltpu.async_copy(idx_hbm.at[base + g + 3], idx_b, sem_ib)

    @pl.when(g + 2 < GPW)
    def _():
      pltpu.make_async_copy(idx_hbm.at[base + g + 2], idx_a, sem_ia).wait()

  plsc.subcore_barrier()
  pltpu.sync_copy(table.at[pl.ds(r0, RPW)], out_hbm.at[cid, pl.ds(r0, RPW)])


@functools.lru_cache(maxsize=1)
def _get_sc_message_pass():
  mesh = plsc.VectorSubcoreMesh(core_axis_name="c", subcore_axis_name="s",
                                num_cores=NC, num_subcores=NS)
  return functools.partial(
      pl.kernel,
      out_type=jax.ShapeDtypeStruct((NC, NPAD, D), jnp.float32),
      mesh=mesh,
      scratch_types=[
          pltpu.VMEM((2 * G, CSZ), jnp.int32),
          pltpu.VMEM((2 * G, CSZ), jnp.int32),
          pltpu.VMEM((CSZ, D), jnp.float32),
          pltpu.VMEM((CSZ, D), jnp.float32),
          pltpu.VMEM((CSZ, D), jnp.float32),
          pltpu.VMEM_SHARED((NPAD, D), jnp.float32),
          pltpu.SemaphoreType.DMA,
          pltpu.SemaphoreType.DMA,
          pltpu.SemaphoreType.DMA,
          pltpu.SemaphoreType.DMA,
          pltpu.SemaphoreType.DMA,
      ],
  )(_msg_body)


R = 2000


def _mlp_block(p0_ref, p1_ref, w1_ref, b1_ref, w2_ref, b2_ref):
  agg = p0_ref[0] + p1_ref[0]
  z = jnp.dot(agg, w1_ref[...], preferred_element_type=jnp.float32)
  z = jnp.maximum(z + b1_ref[...], 0.0)
  o = jnp.dot(z, w2_ref[...], preferred_element_type=jnp.float32)
  return jnp.maximum(o + b2_ref[...], 0.0)


def _mlp_body(p0_ref, p1_ref, w1_ref, b1_ref, w2_ref, b2_ref, o_ref):
  o_ref[...] = _mlp_block(p0_ref, p1_ref, w1_ref, b1_ref, w2_ref, b2_ref)


def _mlp_last_body(p0_ref, p1_ref, w1_ref, b1_ref, w2_ref, b2_ref,
                   wo_ref, bo_ref, o_ref, gmax_ref):
  i = pl.program_id(0)
  o = _mlp_block(p0_ref, p1_ref, w1_ref, b1_ref, w2_ref, b2_ref)
  bm = jnp.max(o, axis=0, keepdims=True)

  @pl.when(i == 0)
  def _():
    gmax_ref[...] = bm

  @pl.when(i > 0)
  def _():
    gmax_ref[...] = jnp.maximum(gmax_ref[...], bm)

  @pl.when(i == pl.num_programs(0) - 1)
  def _():
    g = gmax_ref[...]
    o_ref[...] = (jnp.dot(g, wo_ref[...], preferred_element_type=jnp.float32)
                  + bo_ref[...])


_p_spec0 = pl.BlockSpec((1, R, D), lambda i: (0, i, 0))
_p_spec1 = pl.BlockSpec((1, R, D), lambda i: (1, i, 0))
_h_spec = pl.BlockSpec((R, D), lambda i: (i, 0))
_w1_spec = pl.BlockSpec((D, 2 * D), lambda i: (0, 0))
_b1_spec = pl.BlockSpec((1, 2 * D), lambda i: (0, 0))
_w2_spec = pl.BlockSpec((2 * D, D), lambda i: (0, 0))
_b2_spec = pl.BlockSpec((1, D), lambda i: (0, 0))

_tc_mlp = pl.pallas_call(
    _mlp_body,
    grid=(N // R,),
    in_specs=[_p_spec0, _p_spec1, _w1_spec, _b1_spec, _w2_spec, _b2_spec],
    out_specs=_h_spec,
    out_shape=jax.ShapeDtypeStruct((N, D), jnp.float32),
)

_tc_mlp_last = pl.pallas_call(
    _mlp_last_body,
    grid=(N // R,),
    in_specs=[_p_spec0, _p_spec1, _w1_spec, _b1_spec, _w2_spec, _b2_spec,
              pl.BlockSpec((D, OUT), lambda i: (0, 0)),
              pl.BlockSpec((1, OUT), lambda i: (0, 0))],
    out_specs=pl.BlockSpec((1, OUT), lambda i: (0, 0)),
    out_shape=jax.ShapeDtypeStruct((1, OUT), jnp.float32),
    scratch_shapes=[pltpu.VMEM((1, D), jnp.float32)],
)


def kernel(x, edge_index, W1_0, b1_0, W2_0, b2_0, W1_1, b1_1, W2_1, b2_1,
           W1_2, b1_2, W2_2, b2_2, W_out, b_out):
  src = edge_index[0]
  dst = edge_index[1]
  pad = EPAD - E
  pad_iota = jnp.arange(pad, dtype=jnp.int32)
  srcp = jnp.concatenate([src, pad_iota % N])
  dstp = jnp.concatenate([dst, N + pad_iota % (NPAD - N)])
  idxp = jnp.concatenate(
      [srcp.reshape(NG, G, CSZ), dstp.reshape(NG, G, CSZ)], axis=1)
  zeros = jnp.zeros((NPAD, D), jnp.float32)

  params = [(W1_0, b1_0, W2_0, b2_0), (W1_1, b1_1, W2_1, b2_1),
            (W1_2, b1_2, W2_2, b2_2)]
  sc_message_pass = _get_sc_message_pass()
  h = x
  for l, (W1, b1, W2, b2) in enumerate(params):
    p = sc_message_pass(h, idxp, zeros)
    if l < 2:
      h = _tc_mlp(p, p, W1, b1.reshape(1, -1), W2, b2.reshape(1, -1))
    else:
      out = _tc_mlp_last(p, p, W1, b1.reshape(1, -1), W2,
                         b2.reshape(1, -1), W_out, b_out.reshape(1, -1))
  return out


# speedup vs baseline: 1.0881x; 1.0641x over previous
import functools

import jax
import jax.numpy as jnp
from jax import lax
from jax.experimental import pallas as pl
from jax.experimental.pallas import tpu as pltpu
from jax.experimental.pallas import tpu_sc as plsc

N = 10000
E = 320000
D = 128
OUT = 10

NC = 2
NS = 16
NW = NC * NS
CSZ = 112
CH0 = 96
CH1 = 96
TOTCH = NS * (CH0 + CH1)
EPAD = TOTCH * CSZ
NPAD = 10112
RPW = NPAD // NS

G = 12
GPW = CH0 // G
NG = TOTCH // G


def _msg_body(h_hbm, idx_hbm, z_hbm, out_hbm,
              idx_a, idx_b, rows_a, rows_b, rows_c, table,
              sem_ia, sem_ib, sem_a, sem_b, sem_c):
  cid = lax.axis_index("c")
  sid = lax.axis_index("s")
  r0 = sid * RPW
  base = (cid * NS + sid) * GPW
  @pl.when((cid == 0) & (sid < NS - 1))
  def _():
    pltpu.sync_copy(h_hbm.at[pl.ds(r0, RPW)], table.at[pl.ds(r0, RPW)])

  @pl.when((cid == 0) & (sid == NS - 1))
  def _():
    pltpu.sync_copy(h_hbm.at[pl.ds(r0, N - r0)], table.at[pl.ds(r0, N - r0)])
    pltpu.sync_copy(z_hbm.at[pl.ds(N, NPAD - N)], table.at[pl.ds(N, NPAD - N)])

  @pl.when(cid == 1)
  def _():
    pltpu.sync_copy(z_hbm.at[pl.ds(r0, RPW)], table.at[pl.ds(r0, RPW)])

  plsc.subcore_barrier()

  pltpu.sync_copy(idx_hbm.at[base], idx_a)
  pltpu.async_copy(idx_hbm.at[base + 1], idx_b, sem_ib)
  pltpu.async_copy(h_hbm.at[idx_a.at[0]], rows_a, sem_a)
  pltpu.async_copy(h_hbm.at[idx_a.at[1]], rows_b, sem_b)

  def process(buf, nxt, carry):
    @pl.loop(0, G, step=3)
    def _(i):
      pltpu.make_async_copy(h_hbm.at[buf.at[i]], rows_a, sem_a).wait()
      pltpu.async_copy(h_hbm.at[buf.at[i + 2]], rows_c, sem_c)
      pltpu.sync_copy(rows_a, table.at[buf.at[G + i]], add=True)

      pltpu.make_async_copy(h_hbm.at[buf.at[i + 1]], rows_b, sem_b).wait()

      @pl.when(i + 3 < G)
      def _():
        pltpu.async_copy(h_hbm.at[buf.at[i + 3]], rows_a, sem_a)

      @pl.when((i + 3 >= G) & carry)
      def _():
        pltpu.async_copy(h_hbm.at[nxt.at[0]], rows_a, sem_a)

      pltpu.sync_copy(rows_b, table.at[buf.at[G + i + 1]], add=True)

      pltpu.make_async_copy(h_hbm.at[buf.at[i + 2]], rows_c, sem_c).wait()

      @pl.when(i + 4 < G)
      def _():
        pltpu.async_copy(h_hbm.at[buf.at[i + 4]], rows_b, sem_b)

      @pl.when((i + 4 >= G + 1) & carry)
      def _():
        pltpu.async_copy(h_hbm.at[nxt.at[1]], rows_b, sem_b)

      pltpu.sync_copy(rows_c, table.at[buf.at[G + i + 2]], add=True)

  @pl.loop(0, GPW, step=2)
  def _(g):
    pltpu.make_async_copy(idx_hbm.at[base + g + 1], idx_b, sem_ib).wait()
    process(idx_a, idx_b, True)

    @pl.when(g + 2 < GPW)
    def _():
      pltpu.async_copy(idx_hbm.at[base + g + 2], idx_a, sem_ia)
      pltpu.make_async_copy(idx_hbm.at[base + g + 2], idx_a, sem_ia).wait()

    process(idx_b, idx_a, g + 2 < GPW)

    @pl.when(g + 3 < GPW)
    def _():
      pltpu.async_copy(idx_hbm.at[base + g + 3], idx_b, sem_ib)

  plsc.subcore_barrier()
  pltpu.sync_copy(table.at[pl.ds(r0, RPW)], out_hbm.at[cid, pl.ds(r0, RPW)])


@functools.lru_cache(maxsize=1)
def _get_sc_message_pass():
  mesh = plsc.VectorSubcoreMesh(core_axis_name="c", subcore_axis_name="s",
                                num_cores=NC, num_subcores=NS)
  return functools.partial(
      pl.kernel,
      out_type=jax.ShapeDtypeStruct((NC, NPAD, D), jnp.float32),
      mesh=mesh,
      scratch_types=[
          pltpu.VMEM((2 * G, CSZ), jnp.int32),
          pltpu.VMEM((2 * G, CSZ), jnp.int32),
          pltpu.VMEM((CSZ, D), jnp.float32),
          pltpu.VMEM((CSZ, D), jnp.float32),
          pltpu.VMEM((CSZ, D), jnp.float32),
          pltpu.VMEM_SHARED((NPAD, D), jnp.float32),
          pltpu.SemaphoreType.DMA,
          pltpu.SemaphoreType.DMA,
          pltpu.SemaphoreType.DMA,
          pltpu.SemaphoreType.DMA,
          pltpu.SemaphoreType.DMA,
      ],
  )(_msg_body)


R = 2000


def _mlp_block(p0_ref, p1_ref, w1_ref, b1_ref, w2_ref, b2_ref):
  agg = p0_ref[0] + p1_ref[0]
  z = jnp.dot(agg, w1_ref[...], preferred_element_type=jnp.float32)
  z = jnp.maximum(z + b1_ref[...], 0.0)
  o = jnp.dot(z, w2_ref[...], preferred_element_type=jnp.float32)
  return jnp.maximum(o + b2_ref[...], 0.0)


def _mlp_body(p0_ref, p1_ref, w1_ref, b1_ref, w2_ref, b2_ref, o_ref):
  o_ref[...] = _mlp_block(p0_ref, p1_ref, w1_ref, b1_ref, w2_ref, b2_ref)


def _mlp_last_body(p0_ref, p1_ref, w1_ref, b1_ref, w2_ref, b2_ref,
                   wo_ref, bo_ref, o_ref, gmax_ref):
  i = pl.program_id(0)
  o = _mlp_block(p0_ref, p1_ref, w1_ref, b1_ref, w2_ref, b2_ref)
  bm = jnp.max(o, axis=0, keepdims=True)

  @pl.when(i == 0)
  def _():
    gmax_ref[...] = bm

  @pl.when(i > 0)
  def _():
    gmax_ref[...] = jnp.maximum(gmax_ref[...], bm)

  @pl.when(i == pl.num_programs(0) - 1)
  def _():
    g = gmax_ref[...]
    o_ref[...] = (jnp.dot(g, wo_ref[...], preferred_element_type=jnp.float32)
                  + bo_ref[...])


_p_spec0 = pl.BlockSpec((1, R, D), lambda i: (0, i, 0))
_p_spec1 = pl.BlockSpec((1, R, D), lambda i: (1, i, 0))
_h_spec = pl.BlockSpec((R, D), lambda i: (i, 0))
_w1_spec = pl.BlockSpec((D, 2 * D), lambda i: (0, 0))
_b1_spec = pl.BlockSpec((1, 2 * D), lambda i: (0, 0))
_w2_spec = pl.BlockSpec((2 * D, D), lambda i: (0, 0))
_b2_spec = pl.BlockSpec((1, D), lambda i: (0, 0))

_tc_mlp = pl.pallas_call(
    _mlp_body,
    grid=(N // R,),
    in_specs=[_p_spec0, _p_spec1, _w1_spec, _b1_spec, _w2_spec, _b2_spec],
    out_specs=_h_spec,
    out_shape=jax.ShapeDtypeStruct((N, D), jnp.float32),
)

_tc_mlp_last = pl.pallas_call(
    _mlp_last_body,
    grid=(N // R,),
    in_specs=[_p_spec0, _p_spec1, _w1_spec, _b1_spec, _w2_spec, _b2_spec,
              pl.BlockSpec((D, OUT), lambda i: (0, 0)),
              pl.BlockSpec((1, OUT), lambda i: (0, 0))],
    out_specs=pl.BlockSpec((1, OUT), lambda i: (0, 0)),
    out_shape=jax.ShapeDtypeStruct((1, OUT), jnp.float32),
    scratch_shapes=[pltpu.VMEM((1, D), jnp.float32)],
)


def kernel(x, edge_index, W1_0, b1_0, W2_0, b2_0, W1_1, b1_1, W2_1, b2_1,
           W1_2, b1_2, W2_2, b2_2, W_out, b_out):
  src = edge_index[0]
  dst = edge_index[1]
  pad = EPAD - E
  pad_iota = jnp.arange(pad, dtype=jnp.int32)
  srcp = jnp.concatenate([src, pad_iota % N])
  dstp = jnp.concatenate([dst, N + pad_iota % (NPAD - N)])
  idxp = jnp.concatenate(
      [srcp.reshape(NG, G, CSZ), dstp.reshape(NG, G, CSZ)], axis=1)
  zeros = jnp.zeros((NPAD, D), jnp.float32)

  params = [(W1_0, b1_0, W2_0, b2_0), (W1_1, b1_1, W2_1, b2_1),
            (W1_2, b1_2, W2_2, b2_2)]
  sc_message_pass = _get_sc_message_pass()
  h = x
  for l, (W1, b1, W2, b2) in enumerate(params):
    p = sc_message_pass(h, idxp, zeros)
    if l < 2:
      h = _tc_mlp(p, p, W1, b1.reshape(1, -1), W2, b2.reshape(1, -1))
    else:
      out = _tc_mlp_last(p, p, W1, b1.reshape(1, -1), W2,
                         b2.reshape(1, -1), W_out, b_out.reshape(1, -1))
  return out


# speedup vs baseline: 1.0940x; 1.0054x over previous
import functools

import jax
import jax.numpy as jnp
from jax import lax
from jax.experimental import pallas as pl
from jax.experimental.pallas import tpu as pltpu
from jax.experimental.pallas import tpu_sc as plsc

N = 10000
E = 320000
D = 128
OUT = 10

NC = 2
NS = 16
NW = NC * NS
CSZ = 112
CH0 = 96
CH1 = 96
TOTCH = NS * (CH0 + CH1)
EPAD = TOTCH * CSZ
NPAD = 10112
RPW = NPAD // NS

G = 12
GPW = CH0 // G
NG = TOTCH // G


def _idx_stage(src_hbm, dst_hbm, g, buf, sem):
  pltpu.async_copy(src_hbm.at[g], buf.at[pl.ds(0, G)], sem)
  pltpu.async_copy(dst_hbm.at[g], buf.at[pl.ds(G, G)], sem)


def _idx_wait(src_hbm, dst_hbm, g, buf, sem):
  pltpu.make_async_copy(src_hbm.at[g], buf.at[pl.ds(0, G)], sem).wait()
  pltpu.make_async_copy(dst_hbm.at[g], buf.at[pl.ds(G, G)], sem).wait()


def _msg_body(h_hbm, src_hbm, dst_hbm, z_hbm, out_hbm,
              idx_a, idx_b, rows_a, rows_b, rows_c, table,
              sem_ia, sem_ib, sem_a, sem_b, sem_c):
  cid = lax.axis_index("c")
  sid = lax.axis_index("s")
  r0 = sid * RPW
  base = (cid * NS + sid) * GPW
  @pl.when((cid == 0) & (sid < NS - 1))
  def _():
    pltpu.sync_copy(h_hbm.at[pl.ds(r0, RPW)], table.at[pl.ds(r0, RPW)])

  @pl.when((cid == 0) & (sid == NS - 1))
  def _():
    pltpu.sync_copy(h_hbm.at[pl.ds(r0, N - r0)], table.at[pl.ds(r0, N - r0)])
    pltpu.sync_copy(z_hbm.at[pl.ds(N, NPAD - N)], table.at[pl.ds(N, NPAD - N)])

  @pl.when(cid == 1)
  def _():
    pltpu.sync_copy(z_hbm.at[pl.ds(r0, RPW)], table.at[pl.ds(r0, RPW)])

  plsc.subcore_barrier()

  _idx_stage(src_hbm, dst_hbm, base, idx_a, sem_ia)
  _idx_wait(src_hbm, dst_hbm, base, idx_a, sem_ia)
  _idx_stage(src_hbm, dst_hbm, base + 1, idx_b, sem_ib)
  pltpu.async_copy(h_hbm.at[idx_a.at[0]], rows_a, sem_a)
  pltpu.async_copy(h_hbm.at[idx_a.at[1]], rows_b, sem_b)

  def process(buf, nxt, carry):
    @pl.loop(0, G, step=3)
    def _(i):
      pltpu.make_async_copy(h_hbm.at[buf.at[i]], rows_a, sem_a).wait()
      pltpu.async_copy(h_hbm.at[buf.at[i + 2]], rows_c, sem_c)
      pltpu.sync_copy(rows_a, table.at[buf.at[G + i]], add=True)

      pltpu.make_async_copy(h_hbm.at[buf.at[i + 1]], rows_b, sem_b).wait()

      @pl.when(i + 3 < G)
      def _():
        pltpu.async_copy(h_hbm.at[buf.at[i + 3]], rows_a, sem_a)

      @pl.when((i + 3 >= G) & carry)
      def _():
        pltpu.async_copy(h_hbm.at[nxt.at[0]], rows_a, sem_a)

      pltpu.sync_copy(rows_b, table.at[buf.at[G + i + 1]], add=True)

      pltpu.make_async_copy(h_hbm.at[buf.at[i + 2]], rows_c, sem_c).wait()

      @pl.when(i + 4 < G)
      def _():
        pltpu.async_copy(h_hbm.at[buf.at[i + 4]], rows_b, sem_b)

      @pl.when((i + 4 >= G + 1) & carry)
      def _():
        pltpu.async_copy(h_hbm.at[nxt.at[1]], rows_b, sem_b)

      pltpu.sync_copy(rows_c, table.at[buf.at[G + i + 2]], add=True)

  @pl.loop(0, GPW, step=2)
  def _(g):
    _idx_wait(src_hbm, dst_hbm, base + g + 1, idx_b, sem_ib)
    process(idx_a, idx_b, True)

    @pl.when(g + 2 < GPW)
    def _():
      _idx_stage(src_hbm, dst_hbm, base + g + 2, idx_a, sem_ia)
      _idx_wait(src_hbm, dst_hbm, base + g + 2, idx_a, sem_ia)

    process(idx_b, idx_a, g + 2 < GPW)

    @pl.when(g + 3 < GPW)
    def _():
      _idx_stage(src_hbm, dst_hbm, base + g + 3, idx_b, sem_ib)

  plsc.subcore_barrier()
  pltpu.sync_copy(table.at[pl.ds(r0, RPW)], out_hbm.at[cid, pl.ds(r0, RPW)])


@functools.lru_cache(maxsize=1)
def _get_sc_message_pass():
  mesh = plsc.VectorSubcoreMesh(core_axis_name="c", subcore_axis_name="s",
                                num_cores=NC, num_subcores=NS)
  return functools.partial(
      pl.kernel,
      out_type=jax.ShapeDtypeStruct((NC, NPAD, D), jnp.float32),
      mesh=mesh,
      scratch_types=[
          pltpu.VMEM((2 * G, CSZ), jnp.int32),
          pltpu.VMEM((2 * G, CSZ), jnp.int32),
          pltpu.VMEM((CSZ, D), jnp.float32),
          pltpu.VMEM((CSZ, D), jnp.float32),
          pltpu.VMEM((CSZ, D), jnp.float32),
          pltpu.VMEM_SHARED((NPAD, D), jnp.float32),
          pltpu.SemaphoreType.DMA,
          pltpu.SemaphoreType.DMA,
          pltpu.SemaphoreType.DMA,
          pltpu.SemaphoreType.DMA,
          pltpu.SemaphoreType.DMA,
      ],
  )(_msg_body)


R = 2000


def _mlp_block(p0_ref, p1_ref, w1_ref, b1_ref, w2_ref, b2_ref):
  agg = p0_ref[0] + p1_ref[0]
  z = jnp.dot(agg, w1_ref[...], preferred_element_type=jnp.float32)
  z = jnp.maximum(z + b1_ref[...], 0.0)
  o = jnp.dot(z, w2_ref[...], preferred_element_type=jnp.float32)
  return jnp.maximum(o + b2_ref[...], 0.0)


def _mlp_body(p0_ref, p1_ref, w1_ref, b1_ref, w2_ref, b2_ref, o_ref):
  o_ref[...] = _mlp_block(p0_ref, p1_ref, w1_ref, b1_ref, w2_ref, b2_ref)


def _mlp_last_body(p0_ref, p1_ref, w1_ref, b1_ref, w2_ref, b2_ref,
                   wo_ref, bo_ref, o_ref, gmax_ref):
  i = pl.program_id(0)
  o = _mlp_block(p0_ref, p1_ref, w1_ref, b1_ref, w2_ref, b2_ref)
  bm = jnp.max(o, axis=0, keepdims=True)

  @pl.when(i == 0)
  def _():
    gmax_ref[...] = bm

  @pl.when(i > 0)
  def _():
    gmax_ref[...] = jnp.maximum(gmax_ref[...], bm)

  @pl.when(i == pl.num_programs(0) - 1)
  def _():
    g = gmax_ref[...]
    o_ref[...] = (jnp.dot(g, wo_ref[...], preferred_element_type=jnp.float32)
                  + bo_ref[...])


_p_spec0 = pl.BlockSpec((1, R, D), lambda i: (0, i, 0))
_p_spec1 = pl.BlockSpec((1, R, D), lambda i: (1, i, 0))
_h_spec = pl.BlockSpec((R, D), lambda i: (i, 0))
_w1_spec = pl.BlockSpec((D, 2 * D), lambda i: (0, 0))
_b1_spec = pl.BlockSpec((1, 2 * D), lambda i: (0, 0))
_w2_spec = pl.BlockSpec((2 * D, D), lambda i: (0, 0))
_b2_spec = pl.BlockSpec((1, D), lambda i: (0, 0))

_tc_mlp = pl.pallas_call(
    _mlp_body,
    grid=(N // R,),
    in_specs=[_p_spec0, _p_spec1, _w1_spec, _b1_spec, _w2_spec, _b2_spec],
    out_specs=_h_spec,
    out_shape=jax.ShapeDtypeStruct((N, D), jnp.float32),
)

_tc_mlp_last = pl.pallas_call(
    _mlp_last_body,
    grid=(N // R,),
    in_specs=[_p_spec0, _p_spec1, _w1_spec, _b1_spec, _w2_spec, _b2_spec,
              pl.BlockSpec((D, OUT), lambda i: (0, 0)),
              pl.BlockSpec((1, OUT), lambda i: (0, 0))],
    out_specs=pl.BlockSpec((1, OUT), lambda i: (0, 0)),
    out_shape=jax.ShapeDtypeStruct((1, OUT), jnp.float32),
    scratch_shapes=[pltpu.VMEM((1, D), jnp.float32)],
)


def kernel(x, edge_index, W1_0, b1_0, W2_0, b2_0, W1_1, b1_1, W2_1, b2_1,
           W1_2, b1_2, W2_2, b2_2, W_out, b_out):
  src = edge_index[0]
  dst = edge_index[1]
  pad = EPAD - E
  pad_iota = jnp.arange(pad, dtype=jnp.int32)
  srcp = jnp.concatenate([src, pad_iota % N])
  dstp = jnp.concatenate([dst, N + pad_iota % (NPAD - N)])
  srcp = srcp.reshape(NG, G, CSZ)
  dstp = dstp.reshape(NG, G, CSZ)
  zeros = jnp.zeros((NPAD, D), jnp.float32)

  params = [(W1_0, b1_0, W2_0, b2_0), (W1_1, b1_1, W2_1, b2_1),
            (W1_2, b1_2, W2_2, b2_2)]
  sc_message_pass = _get_sc_message_pass()
  h = x
  for l, (W1, b1, W2, b2) in enumerate(params):
    p = sc_message_pass(h, srcp, dstp, zeros)
    if l < 2:
      h = _tc_mlp(p, p, W1, b1.reshape(1, -1), W2, b2.reshape(1, -1))
    else:
      out = _tc_mlp_last(p, p, W1, b1.reshape(1, -1), W2,
                         b2.reshape(1, -1), W_out, b_out.reshape(1, -1))
  return out
